# baseline (speedup 1.0000x reference)
import jax
import jax.numpy as jnp
from jax.experimental import pallas as pl
from jax.experimental.pallas import tpu as pltpu

_B, _N, _D, _H, _K = 16, 576, 384, 384, 5
_BPP = 2


def _dge_kernel(x_ref, w1_ref, b1_ref, w2_ref, b2_ref, o_ref):
    r = range(_BPP)
    x = [x_ref[i] for i in r]
    norm = [jnp.sqrt(jnp.sum(xi * xi, axis=1, keepdims=True)) for xi in x]
    xn = [x[i] / (norm[i] + 1e-8) for i in r]
    s = [
        jax.lax.dot_general(
            xn[i], xn[i], (((1,), (1,)), ((), ())),
            preferred_element_type=jnp.float32,
        )
        for i in r
    ]
    rowi = jax.lax.broadcasted_iota(jnp.int32, (_N, _N), 0)
    coli = jax.lax.broadcasted_iota(jnp.int32, (_N, _N), 1)
    diag = rowi == coli
    neg_inf = jnp.float32(-jnp.inf)
    s = [jnp.where(diag, neg_inf, si) for si in s]

    v1 = [jnp.max(si, axis=1, keepdims=True) for si in s]
    m = list(v1)
    den = [jnp.ones_like(v) for v in v1]
    for _ in range(_K - 1):
        m = [
            jnp.max(jnp.where(s[i] < m[i], s[i], neg_inf), axis=1,
                    keepdims=True)
            for i in r
        ]
        den = [den[i] + jnp.exp(m[i] - v1[i]) for i in r]

    p = [jnp.where(s[i] >= m[i], jnp.exp(s[i] - v1[i]), 0.0) for i in r]
    bf = jnp.bfloat16
    agg = [
        jnp.dot(p[i].astype(bf), x[i].astype(bf),
                preferred_element_type=jnp.float32) / den[i]
        for i in r
    ]
    h = [x[i] + agg[i] for i in r]
    cdims = (((1,), (1,)), ((), ()))
    w1b = w1_ref[...].astype(bf)
    h = [
        jax.lax.dot_general(hi.astype(bf), w1b, cdims,
                            preferred_element_type=jnp.float32)
        for hi in h
    ]
    h = [jnp.maximum(hi + b1_ref[...], 0.0) for hi in h]
    w2b = w2_ref[...].astype(bf)
    h = [
        jax.lax.dot_general(hi.astype(bf), w2b, cdims,
                            preferred_element_type=jnp.float32)
        for hi in h
    ]
    for i in r:
        o_ref[i] = jnp.maximum(h[i] + b2_ref[...], 0.0)


def kernel(x, W1, b1, W2, b2):
    b1r = b1.reshape(1, _H)
    b2r = b2.reshape(1, _H)
    out = pl.pallas_call(
        _dge_kernel,
        grid=(_B // _BPP,),
        in_specs=[
            pl.BlockSpec((_BPP, _N, _D), lambda b: (b, 0, 0)),
            pl.BlockSpec((_H, _D), lambda b: (0, 0)),
            pl.BlockSpec((1, _H), lambda b: (0, 0)),
            pl.BlockSpec((_H, _H), lambda b: (0, 0)),
            pl.BlockSpec((1, _H), lambda b: (0, 0)),
        ],
        out_specs=pl.BlockSpec((_BPP, _N, _H), lambda b: (b, 0, 0)),
        out_shape=jax.ShapeDtypeStruct((_B, _N, _H), jnp.float32),
        compiler_params=pltpu.CompilerParams(
            dimension_semantics=("parallel",),
        ),
    )(x, W1, b1r, W2, b2r)
    return out


# speedup vs baseline: 1.0543x; 1.0543x over previous
import jax
import jax.numpy as jnp
from jax.experimental import pallas as pl
from jax.experimental.pallas import tpu as pltpu

_B, _N, _D, _H, _K = 16, 576, 384, 384, 5
_BPP = 2


def _dge_kernel(x_ref, w1_ref, b1_ref, w2_ref, b2_ref, o_ref):
    r = range(_BPP)
    x = [x_ref[i] for i in r]
    norm = [jnp.sqrt(jnp.sum(xi * xi, axis=1, keepdims=True)) for xi in x]
    xn = [x[i] / (norm[i] + 1e-8) for i in r]
    s = [
        jax.lax.dot_general(
            xn[i], xn[i], (((1,), (1,)), ((), ())),
            preferred_element_type=jnp.float32,
        )
        for i in r
    ]
    rowi = jax.lax.broadcasted_iota(jnp.int32, (_N, _N), 0)
    coli = jax.lax.broadcasted_iota(jnp.int32, (_N, _N), 1)
    diag = rowi == coli
    neg_inf = jnp.float32(-jnp.inf)
    s = [jnp.where(diag, neg_inf, si) for si in s]

    v1 = [jnp.max(si, axis=1, keepdims=True) for si in s]
    m = list(v1)
    den = [jnp.ones_like(v) for v in v1]
    for _ in range(_K - 1):
        m = [
            jnp.max(jnp.where(s[i] < m[i], s[i], neg_inf), axis=1,
                    keepdims=True)
            for i in r
        ]
        den = [den[i] + jnp.exp(m[i] - v1[i]) for i in r]

    p = [jnp.where(s[i] >= m[i], jnp.exp(s[i] - v1[i]), 0.0) for i in r]
    agg = [
        jnp.dot(p[i], x[i], preferred_element_type=jnp.float32) / den[i]
        for i in r
    ]
    h = [x[i] + agg[i] for i in r]
    cdims = (((1,), (1,)), ((), ()))
    h = [
        jax.lax.dot_general(hi, w1_ref[...], cdims,
                            preferred_element_type=jnp.float32)
        for hi in h
    ]
    h = [jnp.maximum(hi + b1_ref[...], 0.0) for hi in h]
    h = [
        jax.lax.dot_general(hi, w2_ref[...], cdims,
                            preferred_element_type=jnp.float32)
        for hi in h
    ]
    for i in r:
        o_ref[i] = jnp.maximum(h[i] + b2_ref[...], 0.0)


def kernel(x, W1, b1, W2, b2):
    b1r = b1.reshape(1, _H)
    b2r = b2.reshape(1, _H)
    out = pl.pallas_call(
        _dge_kernel,
        grid=(_B // _BPP,),
        in_specs=[
            pl.BlockSpec((_BPP, _N, _D), lambda b: (b, 0, 0)),
            pl.BlockSpec((_H, _D), lambda b: (0, 0)),
            pl.BlockSpec((1, _H), lambda b: (0, 0)),
            pl.BlockSpec((_H, _H), lambda b: (0, 0)),
            pl.BlockSpec((1, _H), lambda b: (0, 0)),
        ],
        out_specs=pl.BlockSpec((_BPP, _N, _H), lambda b: (b, 0, 0)),
        out_shape=jax.ShapeDtypeStruct((_B, _N, _H), jnp.float32),
        compiler_params=pltpu.CompilerParams(
            dimension_semantics=("parallel",),
        ),
    )(x, W1, b1r, W2, b2r)
    return out
